# baseline (speedup 1.0000x reference)
import functools

import jax
import jax.numpy as jnp
import numpy as np
from jax.experimental import pallas as pl

N0 = 10000
E = 320000
D = 128
L = 10
RATIO = 0.8
NP_ = 10240


def _mm_body(x_ref, w_ref, v_ref, h_ref, xv_ref):
    x = x_ref[...]
    h_ref[...] = jnp.dot(x, w_ref[...], preferred_element_type=jnp.float32)
    xv_ref[...] = jnp.dot(x, v_ref[...], preferred_element_type=jnp.float32)


@jax.jit
def _mm(x, w, v):
    return pl.pallas_call(
        _mm_body,
        out_shape=(jax.ShapeDtypeStruct((NP_, D), jnp.float32),
                   jax.ShapeDtypeStruct((NP_, D), jnp.float32)),
    )(x, w, v)


def _pre_body(agg_ref, xv_ref, b_ref, o_ref):
    v = (agg_ref[...] + xv_ref[...]) + b_ref[...]
    o_ref[...] = jnp.maximum(v, 0.0)


@jax.jit
def _pre(agg, xv, b):
    return pl.pallas_call(
        _pre_body,
        out_shape=jax.ShapeDtypeStruct((NP_, D), jnp.float32),
    )(agg, xv, b[None, :])


def _bnscore_body(pre_ref, mean_ref, var_ref, g_ref, be_ref, p_ref, pn_ref,
                  a_ref, xr_ref, xs_ref, sc_ref):
    a = a_ref[0, 0]
    xb = (g_ref[...] * (pre_ref[...] - mean_ref[...])) \
        / jnp.sqrt(var_ref[...] + 1e-5) + be_ref[...]
    xr = jnp.where(xb >= 0, xb, a * xb)
    q = jnp.dot(xr, p_ref[...], preferred_element_type=jnp.float32)
    score = jnp.tanh(q / pn_ref[0, 0])
    xr_ref[...] = xr
    xs_ref[...] = xr * score
    sc_ref[...] = score


@jax.jit
def _bnscore(pre, mean, var, g, be, p, pnorm, a):
    return pl.pallas_call(
        _bnscore_body,
        out_shape=(jax.ShapeDtypeStruct((NP_, D), jnp.float32),
                   jax.ShapeDtypeStruct((NP_, D), jnp.float32),
                   jax.ShapeDtypeStruct((NP_, 1), jnp.float32)),
    )(pre, mean[None, :], var[None, :], g[None, :], be[None, :], p[:, None],
      pnorm.reshape(1, 1), a.reshape(1, 1))


_RB = 1280
_JC = 2048


def _rank_body(n_ref, k_ref, scol_ref, srow_ref, rank_ref, nidx_ref, kept_ref):
    ib = pl.program_id(0)
    n = n_ref[0, 0]
    k = k_ref[0, 0]
    si = scol_ref[...]
    ii = jax.lax.broadcasted_iota(jnp.int32, (_RB, 1), 0) + ib * _RB

    def body(c, acc):
        sj = srow_ref[0, pl.dslice(c * _JC, _JC)][None, :]
        jj = jax.lax.broadcasted_iota(jnp.int32, (1, _JC), 1) + c * _JC
        valid = jj < n
        gt = (sj > si) | ((sj == si) & (jj < ii))
        cnt = jnp.where(gt & valid, 1.0, 0.0)
        return acc + jnp.sum(cnt, axis=1, keepdims=True)

    cnt = jax.lax.fori_loop(0, NP_ // _JC, body, jnp.zeros((_RB, 1), jnp.float32))
    rank = jnp.where(ii < n, cnt.astype(jnp.int32), ii)
    rank_ref[...] = rank
    kept = rank < k
    nidx_ref[...] = jnp.where(kept, rank, 0)
    kept_ref[...] = jnp.where(kept, 1.0, 0.0)


@jax.jit
def _rank(n, k, scol, srow):
    return pl.pallas_call(
        _rank_body,
        grid=(NP_ // _RB,),
        in_specs=[
            pl.BlockSpec((1, 1), lambda i: (0, 0)),
            pl.BlockSpec((1, 1), lambda i: (0, 0)),
            pl.BlockSpec((_RB, 1), lambda i: (i, 0)),
            pl.BlockSpec((1, NP_), lambda i: (0, 0)),
        ],
        out_specs=(pl.BlockSpec((_RB, 1), lambda i: (i, 0)),
                   pl.BlockSpec((_RB, 1), lambda i: (i, 0)),
                   pl.BlockSpec((_RB, 1), lambda i: (i, 0))),
        out_shape=(jax.ShapeDtypeStruct((NP_, 1), jnp.int32),
                   jax.ShapeDtypeStruct((NP_, 1), jnp.int32),
                   jax.ShapeDtypeStruct((NP_, 1), jnp.float32)),
    )(n, k, scol, srow)


def _pool_body(xs_ref, rank_ref, k_ref, o_ref):
    k = k_ref[0, 0]
    mask = rank_ref[...] < k
    xs = xs_ref[...]
    xm = jnp.max(jnp.where(mask, xs, -jnp.inf), axis=0)
    xme = jnp.sum(jnp.where(mask, xs, 0.0), axis=0) / k.astype(jnp.float32)
    o_ref[...] = jnp.concatenate([xm, xme])[None, :]


@jax.jit
def _pool(xs, rank, k):
    return pl.pallas_call(
        _pool_body,
        out_shape=jax.ShapeDtypeStruct((1, 2 * D), jnp.float32),
    )(xs, rank, k)


def _epilogue_body(xc_ref, w1_ref, b1_ref, w2_ref, b2_ref, a_ref, out_ref):
    a = a_ref[0, 0]
    h1 = jnp.dot(xc_ref[...], w1_ref[...], preferred_element_type=jnp.float32)
    h1 = h1 + b1_ref[...]
    h1 = jnp.where(h1 >= 0, h1, a * h1)
    out = jnp.dot(h1, w2_ref[...], preferred_element_type=jnp.float32)
    out = out + b2_ref[...]
    out = jnp.where(out >= 0, out, a * out)
    out = out - jnp.min(out, axis=1, keepdims=True)
    out = out / jnp.max(out, axis=1, keepdims=True)
    out = out / jnp.sum(out, axis=1, keepdims=True)
    out_ref[...] = out


@jax.jit
def _epilogue(xc, w1, b1, w2, b2, a):
    return pl.pallas_call(
        _epilogue_body,
        out_shape=jax.ShapeDtypeStruct((1, 10), jnp.float32),
    )(xc, w1, b1[None, :], w2, b2[None, :], a.reshape(1, 1))


def kernel(x, edge_index, batch, Ws, Vs, bs, gammas, betas, ps,
           lin1_W, lin1_b, lin2_W, lin2_b, prelu_a):
    src = edge_index[0].astype(jnp.int32)
    dst = edge_index[1].astype(jnp.int32)
    emask = jnp.ones((E,), jnp.float32)
    xpad = jnp.zeros((NP_, D), jnp.float32).at[:N0].set(x)
    n = N0
    reads = []
    for i in range(L):
        k = int(np.ceil(RATIO * n))
        deg = jax.ops.segment_sum(emask, dst, num_segments=n)
        dinv = jnp.where(deg > 0, 1.0 / jnp.sqrt(deg), 0.0)
        h, xv = _mm(xpad, Ws[i], Vs[i])
        norm = dinv[src] * dinv[dst] * emask
        upd = h[src] * norm[:, None]
        agg = jax.ops.segment_sum(upd, dst, num_segments=n)
        aggp = jnp.zeros((NP_, D), jnp.float32).at[:n].set(agg)
        pre = _pre(aggp, xv, bs[i])
        mean = pre[:n].mean(axis=0)
        var = pre[:n].var(axis=0)
        pnorm = jnp.linalg.norm(ps[i])
        xr, xs, score = _bnscore(pre, mean, var, gammas[i], betas[i], ps[i],
                                 pnorm, prelu_a)
        rank, nidx, kept = _rank(
            jnp.full((1, 1), n, jnp.int32), jnp.full((1, 1), k, jnp.int32),
            score, score.reshape(1, NP_))
        reads_i = _pool(xs, rank, jnp.full((1, 1), k, jnp.int32))
        reads.append(reads_i)
        xpad = jnp.zeros((NP_, D), jnp.float32).at[rank[:, 0]].set(
            xs, unique_indices=True)
        nidx1 = nidx[:, 0]
        kept1 = kept[:, 0]
        emask = emask * kept1[src] * kept1[dst]
        src = nidx1[src]
        dst = nidx1[dst]
        n = k
    xc = jnp.concatenate(reads, axis=1)
    return _epilogue(xc, lin1_W, lin1_b, lin2_W, lin2_b, prelu_a)
